# baseline (speedup 1.0000x reference)
import functools

import jax
import jax.numpy as jnp
from jax import lax
from jax.experimental import pallas as pl
from jax.experimental.pallas import tpu as pltpu
from jax.experimental.pallas import tpu_sc as plsc

NC = 2
NS = 16
NLANE = 16
NW = NC * NS

ROWS = 128
N = 131072
K = 1024

NBITS = 12
NBINS = 1 << NBITS
SHIFT = 32 - NBITS
HALF = NBINS // 2

ROWS_PER_W = ROWS // NW
CHUNK = 16384
NCHUNK = N // CHUNK
CAND_MAX = 16384

_I32_MIN = -2147483648


def _mono(u):
    return u ^ (lax.shift_right_arithmetic(u, 31) & jnp.int32(0x7FFFFFFF))


def _sc_thresholds(x):
    mesh = plsc.VectorSubcoreMesh(core_axis_name="c", subcore_axis_name="s")

    @functools.partial(
        pl.kernel,
        mesh=mesh,
        out_type=jax.ShapeDtypeStruct((NW, NLANE), jnp.int32),
        compiler_params=pltpu.CompilerParams(needs_layout_passes=False),
        scratch_types=[
            pltpu.VMEM((NLANE * NBINS,), jnp.int32),
            pltpu.VMEM((CHUNK,), jnp.float32),
            pltpu.VMEM((CHUNK,), jnp.float32),
            pltpu.VMEM((CAND_MAX,), jnp.int32),
            pltpu.VMEM((NLANE,), jnp.int32),
            pltpu.SemaphoreType.DMA,
            pltpu.SemaphoreType.DMA,
        ],
    )
    def k(x_hbm, thr_hbm, hist, buf0, buf1, cand, thr_v, sem0, sem1):
        wid = lax.axis_index("s") * NC + lax.axis_index("c")
        iota = lax.iota(jnp.int32, NLANE)
        lane_off = iota * NBINS + HALF
        ones = jnp.ones((NLANE,), jnp.int32)
        zeros16 = jnp.zeros((NLANE,), jnp.int32)

        def dma(row, c, buf, sem):
            return pltpu.make_async_copy(
                x_hbm.at[row, pl.ds(c * CHUNK, CHUNK)], buf, sem)

        def stream_row(row, process, init_carry):
            dma(row, 0, buf0, sem0).start()

            def pair(c2, carry):
                c = c2 * 2
                dma(row, c + 1, buf1, sem1).start()
                dma(row, c, buf0, sem0).wait()
                carry = process(buf0, carry)

                @pl.when(c + 2 < NCHUNK)
                def _():
                    dma(row, c + 2, buf0, sem0).start()
                dma(row, c + 1, buf1, sem1).wait()
                return process(buf1, carry)

            return lax.fori_loop(0, NCHUNK // 2, pair, init_carry)

        @plsc.parallel_loop(0, NLANE * NBINS, NLANE, unroll=8)
        def _(i):
            hist[pl.ds(i, NLANE)] = zeros16

        def row_body(j, thr_vec):
            row = wid * ROWS_PER_W + j

            def p1(buf, carry):
                @plsc.parallel_loop(0, CHUNK, NLANE, unroll=8)
                def _(i):
                    v = buf[pl.ds(i, NLANE)]
                    m = _mono(lax.bitcast_convert_type(v, jnp.int32))
                    bkt = lax.shift_right_arithmetic(m, SHIFT)
                    plsc.addupdate_scatter(hist, [bkt + lane_off], ones)
                return carry
            stream_row(row, p1, jnp.int32(0))

            def scan_body(vb, carry):
                csum, bin_found, count_above = carry
                vbb = NBINS // NLANE - 1 - vb
                base = vbb * NLANE
                tot = hist[pl.ds(base, NLANE)]
                hist[pl.ds(base, NLANE)] = zeros16
                for l in range(1, NLANE):
                    off = l * NBINS + base
                    tot = tot + hist[pl.ds(off, NLANE)]
                    hist[pl.ds(off, NLANE)] = zeros16
                rev = lax.rev(tot, (0,))
                cs = jnp.cumsum(rev)
                s = cs[NLANE - 1]
                mask = cs >= (K - csum)
                nm = jnp.where(mask, 0, 1)
                ffs = jnp.sum(nm)
                cnt_above_in = jnp.sum(jnp.where(mask, 0, rev))
                bin_here = base + (NLANE - 1) - ffs
                crossed = (csum < K) & (csum + s >= K)
                bin_found = jnp.where(crossed, bin_here, bin_found)
                count_above = jnp.where(crossed, csum + cnt_above_in, count_above)
                return csum + s, bin_found, count_above
            _, bin_found, count_above = lax.fori_loop(
                0, NBINS // NLANE, scan_body,
                (jnp.int32(0), jnp.int32(0), jnp.int32(0)))

            rneed = K - count_above
            bin_rel = bin_found - HALF

            def p2(buf, offm1_v):
                @plsc.parallel_loop(0, CHUNK, NLANE, unroll=8, carry=offm1_v)
                def off_out(i, offm1_v):
                    v = buf[pl.ds(i, NLANE)]
                    m = _mono(lax.bitcast_convert_type(v, jnp.int32))
                    is_cand = lax.shift_right_arithmetic(m, SHIFT) == bin_rel
                    cs = jnp.cumsum(is_cand.astype(jnp.int32))
                    plsc.store_scatter(cand, [offm1_v + cs], m, mask=is_cand)
                    pc = plsc.all_reduce_population_count(is_cand)
                    return jnp.minimum(offm1_v + pc, CAND_MAX - 1 - NLANE)
                return off_out
            offm1_v = stream_row(row, p2,
                                 jnp.full((NLANE,), -1, jnp.int32))
            off = offm1_v[0] + 1

            offc = jnp.minimum(off, CAND_MAX - NLANE)
            cand[pl.ds(offc, NLANE)] = jnp.full((NLANE,), _I32_MIN, jnp.int32)
            cnt = jnp.minimum(off, CAND_MAX)
            nv = lax.shift_right_arithmetic(cnt + (NLANE - 1), 4)

            def bs_body(j2, p):
                t = p + lax.shift_left(jnp.int32(1), SHIFT - 1 - j2)

                def cnt_body(i, cv):
                    v = cand[pl.ds(i * NLANE, NLANE)]
                    return cv + (v >= t).astype(jnp.int32)
                cv = lax.fori_loop(0, nv, cnt_body,
                                   jnp.zeros((NLANE,), jnp.int32))
                c = jnp.sum(cv)
                return jnp.where(c >= rneed, t, p)
            p = lax.fori_loop(0, SHIFT, bs_body,
                              lax.shift_left(bin_rel, SHIFT))

            return jnp.where(iota == j, p, thr_vec)

        thr_vec = lax.fori_loop(0, ROWS_PER_W, row_body,
                                jnp.full((NLANE,), _I32_MIN, jnp.int32))
        thr_v[...] = thr_vec
        pltpu.sync_copy(thr_v, thr_hbm.at[wid])

    return k(x)


def _tc_mask(x, thr2d):
    rows_blk = 8
    col_blk = 16384

    def body(x_ref, t_ref, o_ref):
        x = x_ref[...]
        u = lax.bitcast_convert_type(x, jnp.int32)
        m = u ^ (lax.shift_right_arithmetic(u, 31) & jnp.int32(0x7FFFFFFF))
        t = t_ref[:, 0:1]
        o_ref[...] = jnp.where(m >= t, x, jnp.float32(0.0))

    return pl.pallas_call(
        body,
        grid=(ROWS // rows_blk, N // col_blk),
        in_specs=[
            pl.BlockSpec((rows_blk, col_blk), lambda i, j: (i, j)),
            pl.BlockSpec((rows_blk, 128), lambda i, j: (i, 0)),
        ],
        out_specs=pl.BlockSpec((rows_blk, col_blk), lambda i, j: (i, j)),
        out_shape=jax.ShapeDtypeStruct((ROWS, N), jnp.float32),
    )(x, thr2d)


def kernel(features, k):
    batch, n_layers, d_features = features.shape
    flat = features.reshape(batch, n_layers * d_features)
    thr = _sc_thresholds(flat)
    thr128 = thr[:, :ROWS_PER_W].reshape(ROWS)
    thr2d = jnp.broadcast_to(thr128[:, None], (ROWS, 128))
    out = _tc_mask(flat, thr2d)
    return out.reshape(batch, n_layers, d_features)


# speedup vs baseline: 1.3513x; 1.3513x over previous
import functools

import jax
import jax.numpy as jnp
from jax import lax
from jax.experimental import pallas as pl
from jax.experimental.pallas import tpu as pltpu
from jax.experimental.pallas import tpu_sc as plsc

NC = 2
NS = 16
NLANE = 16
NW = NC * NS

ROWS = 128
N = 131072
K = 1024

NBITS = 12
NBINS = 1 << NBITS
SHIFT = 32 - NBITS
HALF = NBINS // 2

ROWS_PER_W = ROWS // NW
CHUNK = 16384
NCHUNK = N // CHUNK
CAND_MAX = 16384

_I32_MIN = -2147483648


def _mono(u):
    return u ^ (lax.shift_right_arithmetic(u, 31) & jnp.int32(0x7FFFFFFF))


def _sc_thresholds(x):
    mesh = plsc.VectorSubcoreMesh(core_axis_name="c", subcore_axis_name="s")

    @functools.partial(
        pl.kernel,
        mesh=mesh,
        out_type=jax.ShapeDtypeStruct((NW, NLANE), jnp.int32),
        compiler_params=pltpu.CompilerParams(needs_layout_passes=False),
        scratch_types=[
            pltpu.VMEM((NLANE * NBINS,), jnp.int32),
            pltpu.VMEM((CHUNK,), jnp.float32),
            pltpu.VMEM((CHUNK,), jnp.float32),
            pltpu.VMEM((CAND_MAX,), jnp.int32),
            pltpu.VMEM((NLANE,), jnp.int32),
            pltpu.SemaphoreType.DMA,
            pltpu.SemaphoreType.DMA,
        ],
    )
    def k(x_hbm, thr_hbm, hist, buf0, buf1, cand, thr_v, sem0, sem1):
        wid = lax.axis_index("s") * NC + lax.axis_index("c")
        iota = lax.iota(jnp.int32, NLANE)
        lane_off = iota * NBINS + HALF
        ones = jnp.ones((NLANE,), jnp.int32)
        zeros16 = jnp.zeros((NLANE,), jnp.int32)

        def dma(row, c, buf, sem):
            return pltpu.make_async_copy(
                x_hbm.at[row, pl.ds(c * CHUNK, CHUNK)], buf, sem)

        def stream_row(row, process, init_carry):
            dma(row, 0, buf0, sem0).start()

            def pair(c2, carry):
                c = c2 * 2
                dma(row, c + 1, buf1, sem1).start()
                dma(row, c, buf0, sem0).wait()
                carry = process(buf0, carry)

                @pl.when(c + 2 < NCHUNK)
                def _():
                    dma(row, c + 2, buf0, sem0).start()
                dma(row, c + 1, buf1, sem1).wait()
                return process(buf1, carry)

            return lax.fori_loop(0, NCHUNK // 2, pair, init_carry)

        @plsc.parallel_loop(0, NLANE * NBINS, NLANE, unroll=8)
        def _(i):
            hist[pl.ds(i, NLANE)] = zeros16

        def row_body(j, thr_vec):
            row = wid * ROWS_PER_W + j

            def p1(buf, carry):
                @plsc.parallel_loop(0, CHUNK, NLANE, unroll=8)
                def _(i):
                    v = buf[pl.ds(i, NLANE)]
                    m = _mono(lax.bitcast_convert_type(v, jnp.int32))
                    bkt = lax.shift_right_arithmetic(m, SHIFT)
                    plsc.addupdate_scatter(hist, [bkt + lane_off], ones)
                return carry
            stream_row(row, p1, jnp.int32(0))

            def scan_body(vb, carry):
                csum, bin_found, count_above = carry
                vbb = NBINS // NLANE - 1 - vb
                base = vbb * NLANE
                tot = hist[pl.ds(base, NLANE)]
                hist[pl.ds(base, NLANE)] = zeros16
                for l in range(1, NLANE):
                    off = l * NBINS + base
                    tot = tot + hist[pl.ds(off, NLANE)]
                    hist[pl.ds(off, NLANE)] = zeros16
                rev = lax.rev(tot, (0,))
                cs = jnp.cumsum(rev)
                s = cs[NLANE - 1]
                mask = cs >= (K - csum)
                nm = jnp.where(mask, 0, 1)
                ffs = jnp.sum(nm)
                cnt_above_in = jnp.sum(jnp.where(mask, 0, rev))
                bin_here = base + (NLANE - 1) - ffs
                crossed = (csum < K) & (csum + s >= K)
                bin_found = jnp.where(crossed, bin_here, bin_found)
                count_above = jnp.where(crossed, csum + cnt_above_in, count_above)
                return csum + s, bin_found, count_above
            _, bin_found, count_above = lax.fori_loop(
                0, NBINS // NLANE, scan_body,
                (jnp.int32(0), jnp.int32(0), jnp.int32(0)))

            rneed = K - count_above
            bin_rel = bin_found - HALF

            def p2(buf, offm1_v):
                @plsc.parallel_loop(0, CHUNK, NLANE, unroll=8, carry=offm1_v)
                def off_out(i, offm1_v):
                    v = buf[pl.ds(i, NLANE)]
                    m = _mono(lax.bitcast_convert_type(v, jnp.int32))
                    is_cand = lax.shift_right_arithmetic(m, SHIFT) == bin_rel
                    cs = jnp.cumsum(is_cand.astype(jnp.int32))
                    plsc.store_scatter(cand, [offm1_v + cs], m, mask=is_cand)
                    pc = plsc.all_reduce_population_count(is_cand)
                    return jnp.minimum(offm1_v + pc, CAND_MAX - 1 - NLANE)
                return off_out
            offm1_v = stream_row(row, p2,
                                 jnp.full((NLANE,), -1, jnp.int32))
            off = offm1_v[0] + 1

            offc = jnp.minimum(off, CAND_MAX - NLANE)
            cand[pl.ds(offc, NLANE)] = jnp.full((NLANE,), _I32_MIN, jnp.int32)
            cnt = jnp.minimum(off, CAND_MAX)
            nv = lax.shift_right_arithmetic(cnt + (NLANE - 1), 4)

            def bs_body(j2, p):
                t = p + lax.shift_left(jnp.int32(1), SHIFT - 1 - j2)

                def cnt_body(i, cv):
                    v = cand[pl.ds(i * NLANE, NLANE)]
                    return cv + (v >= t).astype(jnp.int32)
                cv = lax.fori_loop(0, nv, cnt_body,
                                   jnp.zeros((NLANE,), jnp.int32))
                c = jnp.sum(cv)
                return jnp.where(c >= rneed, t, p)
            p = lax.fori_loop(0, SHIFT, bs_body,
                              lax.shift_left(bin_rel, SHIFT))

            return jnp.where(iota == j, p, thr_vec)

        thr_vec = lax.fori_loop(0, ROWS_PER_W, row_body,
                                jnp.full((NLANE,), _I32_MIN, jnp.int32))
        thr_v[...] = thr_vec
        pltpu.sync_copy(thr_v, thr_hbm.at[wid])

    return k(x)


def _tc_mask(x3, thr2d):
    rows_blk = 8
    n_layers, d_features = x3.shape[1], x3.shape[2]

    def body(x_ref, t_ref, o_ref):
        x = x_ref[...]
        u = lax.bitcast_convert_type(x, jnp.int32)
        m = u ^ (lax.shift_right_arithmetic(u, 31) & jnp.int32(0x7FFFFFFF))
        t = t_ref[:, 0:1].reshape(rows_blk, 1, 1)
        o_ref[...] = jnp.where(m >= t, x, jnp.float32(0.0))

    return pl.pallas_call(
        body,
        grid=(ROWS // rows_blk,),
        in_specs=[
            pl.BlockSpec((rows_blk, n_layers, d_features), lambda i: (i, 0, 0)),
            pl.BlockSpec((rows_blk, 128), lambda i: (i, 0)),
        ],
        out_specs=pl.BlockSpec((rows_blk, n_layers, d_features),
                               lambda i: (i, 0, 0)),
        out_shape=jax.ShapeDtypeStruct((ROWS, n_layers, d_features),
                                       jnp.float32),
    )(x3, thr2d)


def kernel(features, k):
    batch, n_layers, d_features = features.shape
    flat = features.reshape(batch, n_layers * d_features)
    thr = _sc_thresholds(flat)
    thr128 = thr[:, :ROWS_PER_W].reshape(ROWS)
    thr2d = jnp.broadcast_to(thr128[:, None], (ROWS, 128))
    return _tc_mask(features, thr2d)


# speedup vs baseline: 1.3520x; 1.0005x over previous
import functools

import jax
import jax.numpy as jnp
from jax import lax
from jax.experimental import pallas as pl
from jax.experimental.pallas import tpu as pltpu
from jax.experimental.pallas import tpu_sc as plsc

NC = 2
NS = 16
NLANE = 16
NW = NC * NS

ROWS = 128
NLAYERS = 32
DFEAT = 4096
N = NLAYERS * DFEAT
K = 1024

NBITS = 11
NBINS = 1 << NBITS
SHIFT = 32 - NBITS
HALF = NBINS // 2

ROWS_PER_W = ROWS // NW
LAYERS_PER_CHUNK = 8
CHUNK = LAYERS_PER_CHUNK * DFEAT
NCHUNK = N // CHUNK
VREGS_PER_CHUNK = CHUNK // NLANE
CAND_MAX = 16384

_I32_MIN = -2147483648


def _mono(u):
    return u ^ (lax.shift_right_arithmetic(u, 31) & jnp.int32(0x7FFFFFFF))


def _sc_thresholds(x):
    mesh = plsc.VectorSubcoreMesh(core_axis_name="c", subcore_axis_name="s")

    @functools.partial(
        pl.kernel,
        mesh=mesh,
        out_type=jax.ShapeDtypeStruct((NW, NLANE), jnp.int32),
        compiler_params=pltpu.CompilerParams(needs_layout_passes=False,
                                             use_tc_tiling_on_sc=True),
        scratch_types=[
            pltpu.VMEM((NLANE * NBINS,), jnp.int32),
            pltpu.VMEM((LAYERS_PER_CHUNK, DFEAT), jnp.float32),
            pltpu.VMEM((LAYERS_PER_CHUNK, DFEAT), jnp.float32),
            pltpu.VMEM((CAND_MAX,), jnp.int32),
            pltpu.VMEM((NLANE,), jnp.int32),
            pltpu.SemaphoreType.DMA,
            pltpu.SemaphoreType.DMA,
        ],
    )
    def k(x_hbm, thr_hbm, hist, buf0, buf1, cand, thr_v, sem0, sem1):
        wid = lax.axis_index("s") * NC + lax.axis_index("c")
        iota = lax.iota(jnp.int32, NLANE)
        lane_off = iota * NBINS + HALF
        ones = jnp.ones((NLANE,), jnp.int32)
        zeros16 = jnp.zeros((NLANE,), jnp.int32)

        def dma(row, c, buf, sem):
            return pltpu.make_async_copy(
                x_hbm.at[row, pl.ds(c * LAYERS_PER_CHUNK, LAYERS_PER_CHUNK)],
                buf, sem)

        def stream_row(row, process, init_carry):
            dma(row, 0, buf0, sem0).start()

            def pair(c2, carry):
                c = c2 * 2
                dma(row, c + 1, buf1, sem1).start()
                dma(row, c, buf0, sem0).wait()
                carry = process(buf0, carry)

                @pl.when(c + 2 < NCHUNK)
                def _():
                    dma(row, c + 2, buf0, sem0).start()
                dma(row, c + 1, buf1, sem1).wait()
                return process(buf1, carry)

            return lax.fori_loop(0, NCHUNK // 2, pair, init_carry)

        @plsc.parallel_loop(0, NLANE * NBINS, NLANE, unroll=8)
        def _(i):
            hist[pl.ds(i, NLANE)] = zeros16

        def row_body(j, thr_vec):
            row = wid * ROWS_PER_W + j

            def p1(buf, carry):
                for s in range(LAYERS_PER_CHUNK):
                    @plsc.parallel_loop(0, DFEAT, NLANE, unroll=8)
                    def _(i):
                        v = buf[s, pl.ds(i, NLANE)]
                        m = _mono(lax.bitcast_convert_type(v, jnp.int32))
                        bkt = lax.shift_right_arithmetic(m, SHIFT)
                        plsc.addupdate_scatter(hist, [bkt + lane_off], ones)
                return carry
            stream_row(row, p1, jnp.int32(0))

            def scan_body(vb, carry):
                csum, bin_found, count_above = carry
                vbb = NBINS // NLANE - 1 - vb
                base = vbb * NLANE
                tot = hist[pl.ds(base, NLANE)]
                hist[pl.ds(base, NLANE)] = zeros16
                for l in range(1, NLANE):
                    off = l * NBINS + base
                    tot = tot + hist[pl.ds(off, NLANE)]
                    hist[pl.ds(off, NLANE)] = zeros16
                rev = lax.rev(tot, (0,))
                cs = jnp.cumsum(rev)
                s = cs[NLANE - 1]
                mask = cs >= (K - csum)
                nm = jnp.where(mask, 0, 1)
                ffs = jnp.sum(nm)
                cnt_above_in = jnp.sum(jnp.where(mask, 0, rev))
                bin_here = base + (NLANE - 1) - ffs
                crossed = (csum < K) & (csum + s >= K)
                bin_found = jnp.where(crossed, bin_here, bin_found)
                count_above = jnp.where(crossed, csum + cnt_above_in, count_above)
                return csum + s, bin_found, count_above
            _, bin_found, count_above = lax.fori_loop(
                0, NBINS // NLANE, scan_body,
                (jnp.int32(0), jnp.int32(0), jnp.int32(0)))

            rneed = K - count_above
            bin_rel = bin_found - HALF

            def p2(buf, offm1_v):
                for s in range(LAYERS_PER_CHUNK):
                    @plsc.parallel_loop(0, DFEAT, NLANE, unroll=8,
                                        carry=offm1_v)
                    def off_out(i, offm1_v):
                        v = buf[s, pl.ds(i, NLANE)]
                        m = _mono(lax.bitcast_convert_type(v, jnp.int32))
                        is_cand = lax.shift_right_arithmetic(m, SHIFT) == bin_rel
                        cs = jnp.cumsum(is_cand.astype(jnp.int32))
                        plsc.store_scatter(cand, [offm1_v + cs], m,
                                           mask=is_cand)
                        pc = plsc.all_reduce_population_count(is_cand)
                        return jnp.minimum(offm1_v + pc,
                                           CAND_MAX - 1 - NLANE)
                    offm1_v = off_out
                return offm1_v
            offm1_v = stream_row(row, p2,
                                 jnp.full((NLANE,), -1, jnp.int32))
            off = offm1_v[0] + 1

            offc = jnp.minimum(off, CAND_MAX - NLANE)
            cand[pl.ds(offc, NLANE)] = jnp.full((NLANE,), _I32_MIN, jnp.int32)
            cnt = jnp.minimum(off, CAND_MAX)
            nv = lax.shift_right_arithmetic(cnt + (NLANE - 1), 4)

            def bs_body(j2, p):
                t = p + lax.shift_left(jnp.int32(1), SHIFT - 1 - j2)

                def cnt_body(i, cv):
                    v = cand[pl.ds(i * NLANE, NLANE)]
                    return cv + (v >= t).astype(jnp.int32)
                cv = lax.fori_loop(0, nv, cnt_body,
                                   jnp.zeros((NLANE,), jnp.int32))
                c = jnp.sum(cv)
                return jnp.where(c >= rneed, t, p)
            p = lax.fori_loop(0, SHIFT, bs_body,
                              lax.shift_left(bin_rel, SHIFT))

            return jnp.where(iota == j, p, thr_vec)

        thr_vec = lax.fori_loop(0, ROWS_PER_W, row_body,
                                jnp.full((NLANE,), _I32_MIN, jnp.int32))
        thr_v[...] = thr_vec
        pltpu.sync_copy(thr_v, thr_hbm.at[wid])

    return k(x)


def _tc_mask(x3, thr2d):
    rows_blk = 8
    n_layers, d_features = x3.shape[1], x3.shape[2]

    def body(x_ref, t_ref, o_ref):
        x = x_ref[...]
        u = lax.bitcast_convert_type(x, jnp.int32)
        m = u ^ (lax.shift_right_arithmetic(u, 31) & jnp.int32(0x7FFFFFFF))
        t = t_ref[:, 0:1].reshape(rows_blk, 1, 1)
        o_ref[...] = jnp.where(m >= t, x, jnp.float32(0.0))

    return pl.pallas_call(
        body,
        grid=(ROWS // rows_blk,),
        in_specs=[
            pl.BlockSpec((rows_blk, n_layers, d_features), lambda i: (i, 0, 0)),
            pl.BlockSpec((rows_blk, 128), lambda i: (i, 0)),
        ],
        out_specs=pl.BlockSpec((rows_blk, n_layers, d_features),
                               lambda i: (i, 0, 0)),
        out_shape=jax.ShapeDtypeStruct((ROWS, n_layers, d_features),
                                       jnp.float32),
    )(x3, thr2d)


def kernel(features, k):
    thr = _sc_thresholds(features)
    thr128 = thr[:, :ROWS_PER_W].reshape(ROWS)
    thr2d = jnp.broadcast_to(thr128[:, None], (ROWS, 128))
    return _tc_mask(features, thr2d)


# speedup vs baseline: 1.6901x; 1.2501x over previous
import functools

import jax
import jax.numpy as jnp
from jax import lax
from jax.experimental import pallas as pl
from jax.experimental.pallas import tpu as pltpu
from jax.experimental.pallas import tpu_sc as plsc

NC = 2
NS = 16
NLANE = 16
NW = NC * NS

ROWS = 128
NLAYERS = 32
DFEAT = 4096
N = NLAYERS * DFEAT
K = 1024

NBITS = 11
NBINS = 1 << NBITS
SHIFT = 32 - NBITS
HALF = NBINS // 2

ROWS_PER_W = ROWS // NW
LAYERS_PER_CHUNK = 8
CHUNK = LAYERS_PER_CHUNK * DFEAT
NCHUNK = N // CHUNK
VREGS_PER_CHUNK = CHUNK // NLANE
CAND_MAX = 16384

_I32_MIN = -2147483648


def _mono(u):
    return u ^ (lax.shift_right_arithmetic(u, 31) & jnp.int32(0x7FFFFFFF))


def _sc_thresholds(x):
    mesh = plsc.VectorSubcoreMesh(core_axis_name="c", subcore_axis_name="s")

    @functools.partial(
        pl.kernel,
        mesh=mesh,
        out_type=jax.ShapeDtypeStruct((NW, NLANE), jnp.int32),
        compiler_params=pltpu.CompilerParams(needs_layout_passes=False,
                                             use_tc_tiling_on_sc=True),
        scratch_types=[
            pltpu.VMEM((NLANE * NBINS,), jnp.int32),
            pltpu.VMEM((LAYERS_PER_CHUNK, DFEAT), jnp.float32),
            pltpu.VMEM((LAYERS_PER_CHUNK, DFEAT), jnp.float32),
            pltpu.VMEM((CAND_MAX,), jnp.int32),
            pltpu.VMEM((NLANE,), jnp.int32),
            pltpu.SemaphoreType.DMA,
            pltpu.SemaphoreType.DMA,
        ],
    )
    def k(x_hbm, thr_hbm, hist, buf0, buf1, cand, thr_v, sem0, sem1):
        wid = lax.axis_index("s") * NC + lax.axis_index("c")
        iota = lax.iota(jnp.int32, NLANE)
        lane_off = iota * NBINS + HALF
        ones = jnp.ones((NLANE,), jnp.int32)
        zeros16 = jnp.zeros((NLANE,), jnp.int32)

        def dma(row, c, buf, sem):
            return pltpu.make_async_copy(
                x_hbm.at[row, pl.ds(c * LAYERS_PER_CHUNK, LAYERS_PER_CHUNK)],
                buf, sem)

        def stream_row(row, process, init_carry):
            def pair(c2, carry):
                c = c2 * 2
                dma(row, c + 1, buf1, sem1).start()
                dma(row, c, buf0, sem0).wait()
                carry = process(buf0, carry)

                @pl.when(c + 2 < NCHUNK)
                def _():
                    dma(row, c + 2, buf0, sem0).start()
                dma(row, c + 1, buf1, sem1).wait()
                return process(buf1, carry)

            return lax.fori_loop(0, NCHUNK // 2, pair, init_carry)

        @plsc.parallel_loop(0, NLANE * NBINS, NLANE, unroll=8)
        def _(i):
            hist[pl.ds(i, NLANE)] = zeros16

        def row_body(j, thr_vec):
            row = wid * ROWS_PER_W + j

            def p1(buf, carry):
                for s in range(LAYERS_PER_CHUNK):
                    @plsc.parallel_loop(0, DFEAT, NLANE, unroll=8)
                    def _(i):
                        v = buf[s, pl.ds(i, NLANE)]
                        m = _mono(lax.bitcast_convert_type(v, jnp.int32))
                        bkt = lax.shift_right_arithmetic(m, SHIFT)
                        plsc.addupdate_scatter(hist, [bkt + lane_off], ones)
                return carry
            stream_row(row, p1, jnp.int32(0))
            dma(row, 0, buf0, sem0).start()

            def scan_body(vb, carry):
                csum, bin_found, count_above = carry
                vbb = NBINS // NLANE - 1 - vb
                base = vbb * NLANE
                tot = hist[pl.ds(base, NLANE)]
                hist[pl.ds(base, NLANE)] = zeros16
                for l in range(1, NLANE):
                    off = l * NBINS + base
                    tot = tot + hist[pl.ds(off, NLANE)]
                    hist[pl.ds(off, NLANE)] = zeros16
                rev = lax.rev(tot, (0,))
                cs = jnp.cumsum(rev)
                s = cs[NLANE - 1]
                mask = cs >= (K - csum)
                nm = jnp.where(mask, 0, 1)
                ffs = jnp.sum(nm)
                cnt_above_in = jnp.sum(jnp.where(mask, 0, rev))
                bin_here = base + (NLANE - 1) - ffs
                crossed = (csum < K) & (csum + s >= K)
                bin_found = jnp.where(crossed, bin_here, bin_found)
                count_above = jnp.where(crossed, csum + cnt_above_in, count_above)
                return csum + s, bin_found, count_above
            _, bin_found, count_above = lax.fori_loop(
                0, NBINS // NLANE, scan_body,
                (jnp.int32(0), jnp.int32(0), jnp.int32(0)))

            rneed = K - count_above
            bin_rel = bin_found - HALF

            def p2(buf, offm1_v):
                for s in range(LAYERS_PER_CHUNK):
                    @plsc.parallel_loop(0, DFEAT, NLANE, unroll=8,
                                        carry=offm1_v)
                    def off_out(i, offm1_v):
                        v = buf[s, pl.ds(i, NLANE)]
                        m = _mono(lax.bitcast_convert_type(v, jnp.int32))
                        is_cand = lax.shift_right_arithmetic(m, SHIFT) == bin_rel
                        cs = jnp.cumsum(is_cand.astype(jnp.int32))
                        plsc.store_scatter(cand, [offm1_v + cs], m,
                                           mask=is_cand)
                        pc = plsc.all_reduce_population_count(is_cand)
                        return jnp.minimum(offm1_v + pc,
                                           CAND_MAX - 1 - NLANE)
                    offm1_v = off_out
                return offm1_v
            offm1_v = stream_row(row, p2,
                                 jnp.full((NLANE,), -1, jnp.int32))
            off = offm1_v[0] + 1

            @pl.when(j < ROWS_PER_W - 1)
            def _():
                dma(row + 1, 0, buf0, sem0).start()

            offc = jnp.minimum(off, CAND_MAX - NLANE)
            cand[pl.ds(offc, NLANE)] = jnp.full((NLANE,), _I32_MIN, jnp.int32)
            cnt = jnp.minimum(off, CAND_MAX)
            nv = lax.shift_right_arithmetic(cnt + (NLANE - 1), 4)

            def bs_body(j2, p):
                t = p + lax.shift_left(jnp.int32(1), SHIFT - 1 - j2)

                @plsc.parallel_loop(0, nv * NLANE, NLANE, unroll=4,
                                    carry=jnp.zeros((NLANE,), jnp.int32))
                def cv(i, cv):
                    v = cand[pl.ds(i, NLANE)]
                    return cv + (v >= t).astype(jnp.int32)
                c = jnp.sum(cv)
                return jnp.where(c >= rneed, t, p)
            p = lax.fori_loop(0, SHIFT, bs_body,
                              lax.shift_left(bin_rel, SHIFT))

            return jnp.where(iota == j, p, thr_vec)

        dma(wid * ROWS_PER_W, 0, buf0, sem0).start()
        thr_vec = lax.fori_loop(0, ROWS_PER_W, row_body,
                                jnp.full((NLANE,), _I32_MIN, jnp.int32))
        thr_v[...] = thr_vec
        pltpu.sync_copy(thr_v, thr_hbm.at[wid])

    return k(x)


def _tc_mask(x3, thr2d):
    rows_blk = 8
    n_layers, d_features = x3.shape[1], x3.shape[2]

    def body(x_ref, t_ref, o_ref):
        x = x_ref[...]
        u = lax.bitcast_convert_type(x, jnp.int32)
        m = u ^ (lax.shift_right_arithmetic(u, 31) & jnp.int32(0x7FFFFFFF))
        t = t_ref[:, 0:1].reshape(rows_blk, 1, 1)
        o_ref[...] = jnp.where(m >= t, x, jnp.float32(0.0))

    return pl.pallas_call(
        body,
        grid=(ROWS // rows_blk,),
        in_specs=[
            pl.BlockSpec((rows_blk, n_layers, d_features), lambda i: (i, 0, 0)),
            pl.BlockSpec((rows_blk, 128), lambda i: (i, 0)),
        ],
        out_specs=pl.BlockSpec((rows_blk, n_layers, d_features),
                               lambda i: (i, 0, 0)),
        out_shape=jax.ShapeDtypeStruct((ROWS, n_layers, d_features),
                                       jnp.float32),
    )(x3, thr2d)


def kernel(features, k):
    thr = _sc_thresholds(features)
    thr128 = thr[:, :ROWS_PER_W].reshape(ROWS)
    thr2d = jnp.broadcast_to(thr128[:, None], (ROWS, 128))
    return _tc_mask(features, thr2d)


# speedup vs baseline: 1.7190x; 1.0171x over previous
import functools

import jax
import jax.numpy as jnp
from jax import lax
from jax.experimental import pallas as pl
from jax.experimental.pallas import tpu as pltpu
from jax.experimental.pallas import tpu_sc as plsc

NC = 2
NS = 16
NLANE = 16
NW = NC * NS

ROWS = 128
NLAYERS = 32
DFEAT = 4096
N = NLAYERS * DFEAT
K = 1024

NBITS = 12
NBINS = 1 << NBITS
SHIFT = 32 - NBITS
HALF = NBINS // 2

ROWS_PER_W = ROWS // NW
LAYERS_PER_CHUNK = 8
COLS_PER_CHUNK = 2048
CHUNK = LAYERS_PER_CHUNK * COLS_PER_CHUNK
NCHUNK = N // CHUNK
CAND_MAX = 16384

_I32_MIN = -2147483648


def _mono(u):
    return u ^ (lax.shift_right_arithmetic(u, 31) & jnp.int32(0x7FFFFFFF))


def _sc_thresholds(x):
    mesh = plsc.VectorSubcoreMesh(core_axis_name="c", subcore_axis_name="s")

    @functools.partial(
        pl.kernel,
        mesh=mesh,
        out_type=jax.ShapeDtypeStruct((NW, NLANE), jnp.int32),
        compiler_params=pltpu.CompilerParams(needs_layout_passes=False,
                                             use_tc_tiling_on_sc=True),
        scratch_types=[
            pltpu.VMEM((NLANE * NBINS,), jnp.int32),
            pltpu.VMEM((LAYERS_PER_CHUNK, COLS_PER_CHUNK), jnp.float32),
            pltpu.VMEM((LAYERS_PER_CHUNK, COLS_PER_CHUNK), jnp.float32),
            pltpu.VMEM((CAND_MAX,), jnp.int32),
            pltpu.VMEM((NLANE,), jnp.int32),
            pltpu.SemaphoreType.DMA,
            pltpu.SemaphoreType.DMA,
        ],
    )
    def k(x_hbm, thr_hbm, hist, buf0, buf1, cand, thr_v, sem0, sem1):
        wid = lax.axis_index("s") * NC + lax.axis_index("c")
        iota = lax.iota(jnp.int32, NLANE)
        lane_off = iota * NBINS + HALF
        ones = jnp.ones((NLANE,), jnp.int32)
        zeros16 = jnp.zeros((NLANE,), jnp.int32)

        def dma(row, c, buf, sem):
            g = lax.shift_right_arithmetic(c, 1) * LAYERS_PER_CHUNK
            cb = (c & 1) * COLS_PER_CHUNK
            return pltpu.make_async_copy(
                x_hbm.at[row, pl.ds(g, LAYERS_PER_CHUNK),
                         pl.ds(cb, COLS_PER_CHUNK)],
                buf, sem)

        def stream_row(row, process, init_carry):
            def pair(c2, carry):
                c = c2 * 2
                dma(row, c + 1, buf1, sem1).start()
                dma(row, c, buf0, sem0).wait()
                carry = process(buf0, carry)

                @pl.when(c + 2 < NCHUNK)
                def _():
                    dma(row, c + 2, buf0, sem0).start()
                dma(row, c + 1, buf1, sem1).wait()
                return process(buf1, carry)

            return lax.fori_loop(0, NCHUNK // 2, pair, init_carry)

        @plsc.parallel_loop(0, NLANE * NBINS, NLANE, unroll=8)
        def _(i):
            hist[pl.ds(i, NLANE)] = zeros16

        def row_body(j, thr_vec):
            row = wid * ROWS_PER_W + j

            def p1(buf, carry):
                for s in range(LAYERS_PER_CHUNK):
                    @plsc.parallel_loop(0, COLS_PER_CHUNK, NLANE, unroll=8)
                    def _(i):
                        v = buf[s, pl.ds(i, NLANE)]
                        m = _mono(lax.bitcast_convert_type(v, jnp.int32))
                        bkt = lax.shift_right_arithmetic(m, SHIFT)
                        plsc.addupdate_scatter(hist, [bkt + lane_off], ones)
                return carry
            stream_row(row, p1, jnp.int32(0))
            dma(row, 0, buf0, sem0).start()

            def scan_body(vb, carry):
                csum, bin_found, count_above = carry
                vbb = NBINS // NLANE - 1 - vb
                base = vbb * NLANE
                tot = hist[pl.ds(base, NLANE)]
                hist[pl.ds(base, NLANE)] = zeros16
                for l in range(1, NLANE):
                    off = l * NBINS + base
                    tot = tot + hist[pl.ds(off, NLANE)]
                    hist[pl.ds(off, NLANE)] = zeros16
                rev = lax.rev(tot, (0,))
                cs = jnp.cumsum(rev)
                s = cs[NLANE - 1]
                mask = cs >= (K - csum)
                nm = jnp.where(mask, 0, 1)
                ffs = jnp.sum(nm)
                cnt_above_in = jnp.sum(jnp.where(mask, 0, rev))
                bin_here = base + (NLANE - 1) - ffs
                crossed = (csum < K) & (csum + s >= K)
                bin_found = jnp.where(crossed, bin_here, bin_found)
                count_above = jnp.where(crossed, csum + cnt_above_in, count_above)
                return csum + s, bin_found, count_above
            _, bin_found, count_above = lax.fori_loop(
                0, NBINS // NLANE, scan_body,
                (jnp.int32(0), jnp.int32(0), jnp.int32(0)))

            rneed = K - count_above
            bin_rel = bin_found - HALF

            def p2(buf, offm1_v):
                for s in range(LAYERS_PER_CHUNK):
                    @plsc.parallel_loop(0, COLS_PER_CHUNK, NLANE, unroll=8,
                                        carry=offm1_v)
                    def off_out(i, offm1_v):
                        v = buf[s, pl.ds(i, NLANE)]
                        m = _mono(lax.bitcast_convert_type(v, jnp.int32))
                        is_cand = lax.shift_right_arithmetic(m, SHIFT) == bin_rel
                        cs = jnp.cumsum(is_cand.astype(jnp.int32))
                        plsc.store_scatter(cand, [offm1_v + cs], m,
                                           mask=is_cand)
                        pc = plsc.all_reduce_population_count(is_cand)
                        return jnp.minimum(offm1_v + pc,
                                           CAND_MAX - 1 - NLANE)
                    offm1_v = off_out
                return offm1_v
            offm1_v = stream_row(row, p2,
                                 jnp.full((NLANE,), -1, jnp.int32))
            off = offm1_v[0] + 1

            @pl.when(j < ROWS_PER_W - 1)
            def _():
                dma(row + 1, 0, buf0, sem0).start()

            offc = jnp.minimum(off, CAND_MAX - NLANE)
            cand[pl.ds(offc, NLANE)] = jnp.full((NLANE,), _I32_MIN, jnp.int32)
            cnt = jnp.minimum(off, CAND_MAX)
            nv = lax.shift_right_arithmetic(cnt + (NLANE - 1), 4)

            def bs_body(j2, p):
                t = p + lax.shift_left(jnp.int32(1), SHIFT - 1 - j2)

                @plsc.parallel_loop(0, nv * NLANE, NLANE, unroll=4,
                                    carry=jnp.zeros((NLANE,), jnp.int32))
                def cv(i, cv):
                    v = cand[pl.ds(i, NLANE)]
                    return cv + (v >= t).astype(jnp.int32)
                c = jnp.sum(cv)
                return jnp.where(c >= rneed, t, p)
            p = lax.fori_loop(0, SHIFT, bs_body,
                              lax.shift_left(bin_rel, SHIFT))

            return jnp.where(iota == j, p, thr_vec)

        dma(wid * ROWS_PER_W, 0, buf0, sem0).start()
        thr_vec = lax.fori_loop(0, ROWS_PER_W, row_body,
                                jnp.full((NLANE,), _I32_MIN, jnp.int32))
        thr_v[...] = thr_vec
        pltpu.sync_copy(thr_v, thr_hbm.at[wid])

    return k(x)


def _tc_mask(x3, thr2d):
    rows_blk = 8
    n_layers, d_features = x3.shape[1], x3.shape[2]

    def body(x_ref, t_ref, o_ref):
        x = x_ref[...]
        u = lax.bitcast_convert_type(x, jnp.int32)
        m = u ^ (lax.shift_right_arithmetic(u, 31) & jnp.int32(0x7FFFFFFF))
        t = t_ref[:, 0:1].reshape(rows_blk, 1, 1)
        o_ref[...] = jnp.where(m >= t, x, jnp.float32(0.0))

    return pl.pallas_call(
        body,
        grid=(ROWS // rows_blk,),
        in_specs=[
            pl.BlockSpec((rows_blk, n_layers, d_features), lambda i: (i, 0, 0)),
            pl.BlockSpec((rows_blk, 128), lambda i: (i, 0)),
        ],
        out_specs=pl.BlockSpec((rows_blk, n_layers, d_features),
                               lambda i: (i, 0, 0)),
        out_shape=jax.ShapeDtypeStruct((ROWS, n_layers, d_features),
                                       jnp.float32),
    )(x3, thr2d)


def kernel(features, k):
    thr = _sc_thresholds(features)
    thr128 = thr[:, :ROWS_PER_W].reshape(ROWS)
    thr2d = jnp.broadcast_to(thr128[:, None], (ROWS, 128))
    return _tc_mask(features, thr2d)


# speedup vs baseline: 1.7636x; 1.0259x over previous
import functools

import jax
import jax.numpy as jnp
from jax import lax
from jax.experimental import pallas as pl
from jax.experimental.pallas import tpu as pltpu
from jax.experimental.pallas import tpu_sc as plsc

NC = 2
NS = 16
NLANE = 16
NW = NC * NS

ROWS = 128
NLAYERS = 32
DFEAT = 4096
N = NLAYERS * DFEAT
K = 1024

NBITS = 12
NBINS = 1 << NBITS
SHIFT = 32 - NBITS
HALF = NBINS // 2

ROWS_PER_W = ROWS // NW
LAYERS_PER_CHUNK = 8
COLS_PER_CHUNK = 2048
CHUNK = LAYERS_PER_CHUNK * COLS_PER_CHUNK
NCHUNK = N // CHUNK
CAND_MAX = 16384

_I32_MIN = -2147483648


def _mono(u):
    return u ^ (lax.shift_right_arithmetic(u, 31) & jnp.int32(0x7FFFFFFF))


def _sc_thresholds(x):
    mesh = plsc.VectorSubcoreMesh(core_axis_name="c", subcore_axis_name="s")

    @functools.partial(
        pl.kernel,
        mesh=mesh,
        out_type=jax.ShapeDtypeStruct((NW, NLANE), jnp.int32),
        compiler_params=pltpu.CompilerParams(needs_layout_passes=False,
                                             use_tc_tiling_on_sc=True),
        scratch_types=[
            pltpu.VMEM((NBINS,), jnp.int32),
            pltpu.VMEM((LAYERS_PER_CHUNK, COLS_PER_CHUNK), jnp.float32),
            pltpu.VMEM((LAYERS_PER_CHUNK, COLS_PER_CHUNK), jnp.float32),
            pltpu.VMEM((CAND_MAX,), jnp.int32),
            pltpu.VMEM((NLANE,), jnp.int32),
            pltpu.SemaphoreType.DMA,
            pltpu.SemaphoreType.DMA,
        ],
    )
    def k(x_hbm, thr_hbm, hist, buf0, buf1, cand, thr_v, sem0, sem1):
        wid = lax.axis_index("s") * NC + lax.axis_index("c")
        iota = lax.iota(jnp.int32, NLANE)
        half_vec = jnp.full((NLANE,), HALF, jnp.int32)
        ones = jnp.ones((NLANE,), jnp.int32)
        zeros16 = jnp.zeros((NLANE,), jnp.int32)

        def dma(row, c, buf, sem):
            g = lax.shift_right_arithmetic(c, 1) * LAYERS_PER_CHUNK
            cb = (c & 1) * COLS_PER_CHUNK
            return pltpu.make_async_copy(
                x_hbm.at[row, pl.ds(g, LAYERS_PER_CHUNK),
                         pl.ds(cb, COLS_PER_CHUNK)],
                buf, sem)

        def stream_row(row, process, init_carry):
            def pair(c2, carry):
                c = c2 * 2
                dma(row, c + 1, buf1, sem1).start()
                dma(row, c, buf0, sem0).wait()
                carry = process(buf0, carry)

                @pl.when(c + 2 < NCHUNK)
                def _():
                    dma(row, c + 2, buf0, sem0).start()
                dma(row, c + 1, buf1, sem1).wait()
                return process(buf1, carry)

            return lax.fori_loop(0, NCHUNK // 2, pair, init_carry)

        @plsc.parallel_loop(0, NBINS, NLANE, unroll=8)
        def _(i):
            hist[pl.ds(i, NLANE)] = zeros16

        def row_body(j, thr_vec):
            row = wid * ROWS_PER_W + j

            def p1(buf, carry):
                for s in range(LAYERS_PER_CHUNK):
                    @plsc.parallel_loop(0, COLS_PER_CHUNK, NLANE, unroll=8)
                    def _(i):
                        v = buf[s, pl.ds(i, NLANE)]
                        m = _mono(lax.bitcast_convert_type(v, jnp.int32))
                        bkt = lax.shift_right_arithmetic(m, SHIFT)
                        plsc.addupdate_scatter(hist, [bkt + half_vec], ones)
                return carry
            stream_row(row, p1, jnp.int32(0))
            dma(row, 0, buf0, sem0).start()

            def scan_body(vb, carry):
                csum, bin_found, count_above = carry
                vbb = NBINS // NLANE - 1 - vb
                base = vbb * NLANE
                tot = hist[pl.ds(base, NLANE)]
                hist[pl.ds(base, NLANE)] = zeros16
                rev = lax.rev(tot, (0,))
                cs = jnp.cumsum(rev)
                s = cs[NLANE - 1]
                mask = cs >= (K - csum)
                nm = jnp.where(mask, 0, 1)
                ffs = jnp.sum(nm)
                cnt_above_in = jnp.sum(jnp.where(mask, 0, rev))
                bin_here = base + (NLANE - 1) - ffs
                crossed = (csum < K) & (csum + s >= K)
                bin_found = jnp.where(crossed, bin_here, bin_found)
                count_above = jnp.where(crossed, csum + cnt_above_in, count_above)
                return csum + s, bin_found, count_above
            _, bin_found, count_above = lax.fori_loop(
                0, NBINS // NLANE, scan_body,
                (jnp.int32(0), jnp.int32(0), jnp.int32(0)))

            rneed = K - count_above
            bin_rel = bin_found - HALF

            def p2(buf, offm1_v):
                for s in range(LAYERS_PER_CHUNK):
                    @plsc.parallel_loop(0, COLS_PER_CHUNK, NLANE, unroll=8,
                                        carry=offm1_v)
                    def off_out(i, offm1_v):
                        v = buf[s, pl.ds(i, NLANE)]
                        m = _mono(lax.bitcast_convert_type(v, jnp.int32))
                        is_cand = lax.shift_right_arithmetic(m, SHIFT) == bin_rel
                        cs = jnp.cumsum(is_cand.astype(jnp.int32))
                        plsc.store_scatter(cand, [offm1_v + cs], m,
                                           mask=is_cand)
                        pc = plsc.all_reduce_population_count(is_cand)
                        return jnp.minimum(offm1_v + pc,
                                           CAND_MAX - 1 - NLANE)
                    offm1_v = off_out
                return offm1_v
            offm1_v = stream_row(row, p2,
                                 jnp.full((NLANE,), -1, jnp.int32))
            off = offm1_v[0] + 1

            @pl.when(j < ROWS_PER_W - 1)
            def _():
                dma(row + 1, 0, buf0, sem0).start()

            offc = jnp.minimum(off, CAND_MAX - NLANE)
            cand[pl.ds(offc, NLANE)] = jnp.full((NLANE,), _I32_MIN, jnp.int32)
            cnt = jnp.minimum(off, CAND_MAX)
            nv = lax.shift_right_arithmetic(cnt + (NLANE - 1), 4)

            def bs_body(j2, p):
                t = p + lax.shift_left(jnp.int32(1), SHIFT - 1 - j2)

                @plsc.parallel_loop(0, nv * NLANE, NLANE, unroll=4,
                                    carry=jnp.zeros((NLANE,), jnp.int32))
                def cv(i, cv):
                    v = cand[pl.ds(i, NLANE)]
                    return cv + (v >= t).astype(jnp.int32)
                c = jnp.sum(cv)
                return jnp.where(c >= rneed, t, p)
            p = lax.fori_loop(0, SHIFT, bs_body,
                              lax.shift_left(bin_rel, SHIFT))

            return jnp.where(iota == j, p, thr_vec)

        dma(wid * ROWS_PER_W, 0, buf0, sem0).start()
        thr_vec = lax.fori_loop(0, ROWS_PER_W, row_body,
                                jnp.full((NLANE,), _I32_MIN, jnp.int32))
        thr_v[...] = thr_vec
        pltpu.sync_copy(thr_v, thr_hbm.at[wid])

    return k(x)


def _tc_mask(x3, thr2d):
    rows_blk = 8
    n_layers, d_features = x3.shape[1], x3.shape[2]

    def body(x_ref, t_ref, o_ref):
        x = x_ref[...]
        u = lax.bitcast_convert_type(x, jnp.int32)
        m = u ^ (lax.shift_right_arithmetic(u, 31) & jnp.int32(0x7FFFFFFF))
        t = t_ref[:, 0:1].reshape(rows_blk, 1, 1)
        o_ref[...] = jnp.where(m >= t, x, jnp.float32(0.0))

    return pl.pallas_call(
        body,
        grid=(ROWS // rows_blk,),
        in_specs=[
            pl.BlockSpec((rows_blk, n_layers, d_features), lambda i: (i, 0, 0)),
            pl.BlockSpec((rows_blk, 128), lambda i: (i, 0)),
        ],
        out_specs=pl.BlockSpec((rows_blk, n_layers, d_features),
                               lambda i: (i, 0, 0)),
        out_shape=jax.ShapeDtypeStruct((ROWS, n_layers, d_features),
                                       jnp.float32),
    )(x3, thr2d)


def kernel(features, k):
    thr = _sc_thresholds(features)
    thr128 = thr[:, :ROWS_PER_W].reshape(ROWS)
    thr2d = jnp.broadcast_to(thr128[:, None], (ROWS, 128))
    return _tc_mask(features, thr2d)


# speedup vs baseline: 1.8083x; 1.0253x over previous
import functools

import jax
import jax.numpy as jnp
from jax import lax
from jax.experimental import pallas as pl
from jax.experimental.pallas import tpu as pltpu
from jax.experimental.pallas import tpu_sc as plsc

NC = 2
NS = 16
NLANE = 16
NW = NC * NS

ROWS = 128
NLAYERS = 32
DFEAT = 4096
N = NLAYERS * DFEAT
K = 1024

NBITS = 12
NBINS = 1 << NBITS
SHIFT = 32 - NBITS
HALF = NBINS // 2

ROWS_PER_W = ROWS // NW
LAYERS_PER_CHUNK = 8
COLS_PER_CHUNK = 4096
CHUNK = LAYERS_PER_CHUNK * COLS_PER_CHUNK
NCHUNK = N // CHUNK
CAND_MAX = 16384

_I32_MIN = -2147483648


def _mono(u):
    return u ^ (lax.shift_right_arithmetic(u, 31) & jnp.int32(0x7FFFFFFF))


def _sc_thresholds(x):
    mesh = plsc.VectorSubcoreMesh(core_axis_name="c", subcore_axis_name="s")

    @functools.partial(
        pl.kernel,
        mesh=mesh,
        out_type=jax.ShapeDtypeStruct((NW, NLANE), jnp.int32),
        compiler_params=pltpu.CompilerParams(needs_layout_passes=False,
                                             use_tc_tiling_on_sc=True),
        scratch_types=[
            pltpu.VMEM((NBINS,), jnp.int32),
            pltpu.VMEM((LAYERS_PER_CHUNK, COLS_PER_CHUNK), jnp.float32),
            pltpu.VMEM((LAYERS_PER_CHUNK, COLS_PER_CHUNK), jnp.float32),
            pltpu.VMEM((CAND_MAX,), jnp.int32),
            pltpu.VMEM((NLANE,), jnp.int32),
            pltpu.SemaphoreType.DMA,
            pltpu.SemaphoreType.DMA,
        ],
    )
    def k(x_hbm, thr_hbm, hist, buf0, buf1, cand, thr_v, sem0, sem1):
        wid = lax.axis_index("s") * NC + lax.axis_index("c")
        iota = lax.iota(jnp.int32, NLANE)
        half_vec = jnp.full((NLANE,), HALF, jnp.int32)
        ones = jnp.ones((NLANE,), jnp.int32)
        zeros16 = jnp.zeros((NLANE,), jnp.int32)

        def dma(row, c, buf, sem):
            return pltpu.make_async_copy(
                x_hbm.at[row, pl.ds(c * LAYERS_PER_CHUNK, LAYERS_PER_CHUNK)],
                buf, sem)

        def stream_row(row, process, init_carry):
            def pair(c2, carry):
                c = c2 * 2
                dma(row, c + 1, buf1, sem1).start()
                dma(row, c, buf0, sem0).wait()
                carry = process(buf0, carry)

                @pl.when(c + 2 < NCHUNK)
                def _():
                    dma(row, c + 2, buf0, sem0).start()
                dma(row, c + 1, buf1, sem1).wait()
                return process(buf1, carry)

            return lax.fori_loop(0, NCHUNK // 2, pair, init_carry)

        @plsc.parallel_loop(0, NBINS, NLANE, unroll=8)
        def _(i):
            hist[pl.ds(i, NLANE)] = zeros16

        def row_body(j, thr_vec):
            row = wid * ROWS_PER_W + j

            def p1(buf, carry):
                for s in range(LAYERS_PER_CHUNK):
                    @plsc.parallel_loop(0, COLS_PER_CHUNK, NLANE, unroll=8)
                    def _(i):
                        v = buf[s, pl.ds(i, NLANE)]
                        m = _mono(lax.bitcast_convert_type(v, jnp.int32))
                        bkt = lax.shift_right_arithmetic(m, SHIFT)
                        plsc.addupdate_scatter(hist, [bkt + half_vec], ones)
                return carry
            stream_row(row, p1, jnp.int32(0))
            dma(row, 0, buf0, sem0).start()

            def scan_body(vb, carry):
                csum, bin_found, count_above = carry
                vbb = NBINS // NLANE - 1 - vb
                base = vbb * NLANE
                tot = hist[pl.ds(base, NLANE)]
                hist[pl.ds(base, NLANE)] = zeros16
                rev = lax.rev(tot, (0,))
                cs = jnp.cumsum(rev)
                s = cs[NLANE - 1]
                mask = cs >= (K - csum)
                nm = jnp.where(mask, 0, 1)
                ffs = jnp.sum(nm)
                cnt_above_in = jnp.sum(jnp.where(mask, 0, rev))
                bin_here = base + (NLANE - 1) - ffs
                crossed = (csum < K) & (csum + s >= K)
                bin_found = jnp.where(crossed, bin_here, bin_found)
                count_above = jnp.where(crossed, csum + cnt_above_in, count_above)
                return csum + s, bin_found, count_above
            _, bin_found, count_above = lax.fori_loop(
                0, NBINS // NLANE, scan_body,
                (jnp.int32(0), jnp.int32(0), jnp.int32(0)))

            rneed = K - count_above
            bin_rel = bin_found - HALF

            def p2(buf, offm1_v):
                for s in range(LAYERS_PER_CHUNK):
                    @plsc.parallel_loop(0, COLS_PER_CHUNK, NLANE, unroll=8,
                                        carry=offm1_v)
                    def off_out(i, offm1_v):
                        v = buf[s, pl.ds(i, NLANE)]
                        m = _mono(lax.bitcast_convert_type(v, jnp.int32))
                        is_cand = lax.shift_right_arithmetic(m, SHIFT) == bin_rel
                        cs = jnp.cumsum(is_cand.astype(jnp.int32))
                        plsc.store_scatter(cand, [offm1_v + cs], m,
                                           mask=is_cand)
                        pc = plsc.all_reduce_population_count(is_cand)
                        return jnp.minimum(offm1_v + pc,
                                           CAND_MAX - 1 - NLANE)
                    offm1_v = off_out
                return offm1_v
            offm1_v = stream_row(row, p2,
                                 jnp.full((NLANE,), -1, jnp.int32))
            off = offm1_v[0] + 1

            @pl.when(j < ROWS_PER_W - 1)
            def _():
                dma(row + 1, 0, buf0, sem0).start()

            offc = jnp.minimum(off, CAND_MAX - NLANE)
            cand[pl.ds(offc, NLANE)] = jnp.full((NLANE,), _I32_MIN, jnp.int32)
            cnt = jnp.minimum(off, CAND_MAX)
            nv = lax.shift_right_arithmetic(cnt + (NLANE - 1), 4)

            def bs_body(j2, p):
                t = p + lax.shift_left(jnp.int32(1), SHIFT - 1 - j2)

                @plsc.parallel_loop(0, nv * NLANE, NLANE, unroll=4,
                                    carry=jnp.zeros((NLANE,), jnp.int32))
                def cv(i, cv):
                    v = cand[pl.ds(i, NLANE)]
                    return cv + (v >= t).astype(jnp.int32)
                c = jnp.sum(cv)
                return jnp.where(c >= rneed, t, p)
            p = lax.fori_loop(0, SHIFT, bs_body,
                              lax.shift_left(bin_rel, SHIFT))

            return jnp.where(iota == j, p, thr_vec)

        dma(wid * ROWS_PER_W, 0, buf0, sem0).start()
        thr_vec = lax.fori_loop(0, ROWS_PER_W, row_body,
                                jnp.full((NLANE,), _I32_MIN, jnp.int32))
        thr_v[...] = thr_vec
        pltpu.sync_copy(thr_v, thr_hbm.at[wid])

    return k(x)


def _tc_mask(x3, thr2d):
    rows_blk = 8
    n_layers, d_features = x3.shape[1], x3.shape[2]

    def body(x_ref, t_ref, o_ref):
        x = x_ref[...]
        u = lax.bitcast_convert_type(x, jnp.int32)
        m = u ^ (lax.shift_right_arithmetic(u, 31) & jnp.int32(0x7FFFFFFF))
        i = pl.program_id(0)
        t8 = t_ref[pl.ds(2 * i, 2), 0:ROWS_PER_W]
        t = t8.reshape(rows_blk, 1, 1)
        o_ref[...] = jnp.where(m >= t, x, jnp.float32(0.0))

    return pl.pallas_call(
        body,
        grid=(ROWS // rows_blk,),
        in_specs=[
            pl.BlockSpec((rows_blk, n_layers, d_features), lambda i: (i, 0, 0)),
            pl.BlockSpec((NW, NLANE), lambda i: (0, 0)),
        ],
        out_specs=pl.BlockSpec((rows_blk, n_layers, d_features),
                               lambda i: (i, 0, 0)),
        out_shape=jax.ShapeDtypeStruct((ROWS, n_layers, d_features),
                                       jnp.float32),
    )(x3, thr2d)


def kernel(features, k):
    thr = _sc_thresholds(features)
    return _tc_mask(features, thr)
